# baseline (speedup 1.0000x reference)
import jax
import jax.numpy as jnp
from jax.experimental import pallas as pl
from jax.experimental.pallas import tpu as pltpu

_EPS = 1e-5
_B = 8
_T = 20
_H = 224
_W = 224
_C = 1536
_NH = 12
_HD = 128
_NG = 5
_K = 5
_TB = 4
_NP = _B * (_T // _TB)
_CQ = 192
_NCQ = 3 * _C // _CQ
_NCS = 12
_IT0 = 25


def _attn_iter(cl_ref, gr_ref, wqkv_bf, wproj_bf, wtfc_bf,
               lnw, lnb, bp_ref, bt_ref):
    ncls = _B * _NG
    ntok = _B * _T + ncls
    ri = jax.lax.broadcasted_iota(jnp.int32, (ntok, ntok), 0)
    ci = jax.lax.broadcasted_iota(jnp.int32, (ntok, ntok), 1)
    gid_r = jnp.where(ri < ncls, ri, jax.lax.div(ri - ncls, 4))
    gid_c = jnp.where(ci < ncls, ci, jax.lax.div(ci - ncls, 4))
    mf = (gid_r == gid_c).astype(jnp.float32)
    scale = float(_HD) ** -0.5
    ones_col = jnp.ones((ntok, 1), jnp.float32)
    tok = jnp.concatenate([cl_ref[...], gr_ref[...]], axis=0)
    mu = jnp.mean(tok, axis=1, keepdims=True)
    d = tok - mu
    var = jnp.mean(d * d, axis=1, keepdims=True)
    h = ((d * jax.lax.rsqrt(var + _EPS)) * lnw + lnb)
    qkv = jax.lax.dot_general(
        h.astype(jnp.bfloat16), wqkv_bf[...], (((1,), (1,)), ((), ())),
        preferred_element_type=jnp.float32)
    oparts = []
    for hh in range(_NH):
        qh = qkv[:, hh * _HD:(hh + 1) * _HD]
        kh = qkv[:, _C + hh * _HD:_C + (hh + 1) * _HD]
        vh = qkv[:, 2 * _C + hh * _HD:2 * _C + (hh + 1) * _HD]
        s = jax.lax.dot_general(
            qh, kh, (((1,), (1,)), ((), ())),
            preferred_element_type=jnp.float32)
        e = jnp.exp(s * scale) * mf
        vaug = jnp.concatenate([vh, ones_col], axis=1)
        oa = jax.lax.dot_general(
            e, vaug, (((1,), (0,)), ((), ())),
            preferred_element_type=jnp.float32)
        oparts.append(oa[:, :_HD] * (1.0 / oa[:, _HD:_HD + 1]))
    o = jnp.concatenate(oparts, axis=1)
    p = jax.lax.dot_general(
        o.astype(jnp.bfloat16), wproj_bf[...], (((1,), (1,)), ((), ())),
        preferred_element_type=jnp.float32) + bp_ref[...]
    u = jax.lax.dot_general(
        p.astype(jnp.bfloat16), wtfc_bf[...], (((1,), (1,)), ((), ())),
        preferred_element_type=jnp.float32) + bt_ref[...]
    upd = u + tok
    cl_new = upd[:ncls].reshape(_B, _NG, _C)
    cl_ref[...] = jnp.concatenate(
        [cl_new[:, _NG - 1:_NG, :], cl_new[:, :_NG - 1, :]],
        axis=1).reshape(ncls, _C)
    gr_ref[...] = upd[ncls:]


def _mega_kernel(bb_ref, ts_ref, x_ref, f_ref, wr_ref, brd_ref, te_ref,
                 wn_ref, bn_ref, cls_ref, lnw_ref, lnb_ref, wqkv_ref,
                 wproj_ref, bp_ref, wtfc_ref, bt_ref,
                 o_ref, g_ref, pooled_ref,
                 wqkv_bf, wproj_bf, wtfc_bf, xt_s, cl_ref, gr_ref):
    i = pl.program_id(0)

    @pl.when(i < _NP)
    def _pool():
        b = i // (_T // _TB)
        tb = jax.lax.rem(i, _T // _TB)
        ri = jax.lax.broadcasted_iota(jnp.int32, (_H, _W), 0)
        ci = jax.lax.broadcasted_iota(jnp.int32, (_H, _W), 1)
        inv_hw = 1.0 / float(_H * _W)
        for tt in range(_TB):
            base = (b * _T + tb * _TB + tt) * 4
            x1 = bb_ref[base]
            y1 = bb_ref[base + 1]
            x2 = bb_ref[base + 2]
            y2 = bb_ref[base + 3]
            mask = (ri >= y1) & (ri < y2) & (ci >= x1) & (ci < x2)
            area = ((x2 - x1) * (y2 - y1)).astype(jnp.float32)
            inv_area = 1.0 / area
            for c in range(3):
                img = x_ref[0, c, tt]
                tot = jnp.sum(img, keepdims=True)
                crop = jnp.sum(jnp.where(mask, img, 0.0), keepdims=True)
                val = tot * inv_hw + crop * inv_area
                o_ref[0, tt, c] = jnp.broadcast_to(val, (1, 128))[0]

    @pl.when(i < _NCQ)
    def _stream_qkv():
        r = pl.multiple_of(i * _CQ, _CQ)
        wqkv_bf[pl.ds(r, _CQ), :] = wqkv_ref[...].astype(jnp.bfloat16)

    @pl.when(i < _NCS)
    def _stream_sq():
        r = pl.multiple_of(i * _HD, _HD)
        wproj_bf[pl.ds(r, _HD), :] = wproj_ref[...].astype(jnp.bfloat16)
        wtfc_bf[pl.ds(r, _HD), :] = wtfc_ref[...].astype(jnp.bfloat16)
        xt_s[:, pl.ds(r, _HD)] = jax.lax.dot_general(
            f_ref[...].reshape(_B * _T, _C), wr_ref[...],
            (((1,), (1,)), ((), ())),
            preferred_element_type=jnp.float32)

    @pl.when(i == _NCS)
    def _embed():
        ts = jnp.stack([ts_ref[b] for b in range(_B)]).reshape(_B, 1)
        tsadd = ts * wn_ref[...] + bn_ref[...]
        x3 = (xt_s[...].reshape(_B, _T, _C) + te_ref[...][None, :, :]
              + brd_ref[...][None, :, :] + tsadd[:, None, :])
        pooled_ref[...] = (x3[:, _T - 2, :] + x3[:, _T - 1, :]) * (1.0 / 3.0)
        gr_ref[...] = x3.reshape(_B * _T, _C)
        cl_ref[...] = jnp.broadcast_to(cls_ref[...][None, :, :],
                                       (_B, _NG, _C)).reshape(_B * _NG, _C)

    for it in range(4):
        @pl.when(i == _IT0 + 4 * it)
        def _iter():
            _attn_iter(cl_ref, gr_ref, wqkv_bf, wproj_bf, wtfc_bf,
                       lnw_ref[...], lnb_ref[...], bp_ref, bt_ref)

    @pl.when(i == _NP)
    def _finish():
        g_ref[:, :, 0:1, :] = cl_ref[...].reshape(_B, _NG, 1, _C)
        g_ref[:, :, 1:5, :] = gr_ref[...].reshape(_B, _NG, 4, _C)


def kernel(x, timestamp, bboxes, feats, w_reduce, b_reduce, w_num, b_num,
           time_embed, cls_token_swap, ln_t_w, ln_t_b, w_qkv, w_proj, b_proj,
           w_tfc, b_tfc):
    bb_flat = bboxes.reshape(-1)
    brd = b_reduce.reshape(1, _C)
    bnm = b_num.reshape(1, _C)
    wnr = w_num.reshape(1, _C)
    te = time_embed.reshape(_T, _C)
    cls2 = cls_token_swap.reshape(_NG, _C)
    lnw = ln_t_w.reshape(1, _C)
    lnb = ln_t_b.reshape(1, _C)
    bpj = b_proj.reshape(1, _C)
    btf = b_tfc.reshape(1, _C)

    npool = _T // _TB
    pool_out, g, pooled = pl.pallas_call(
        _mega_kernel,
        out_shape=[jax.ShapeDtypeStruct((_B, _T, 3, 128), jnp.float32),
                   jax.ShapeDtypeStruct((_B, _NG, _K, _C), jnp.float32),
                   jax.ShapeDtypeStruct((_B, _C), jnp.float32)],
        grid_spec=pltpu.PrefetchScalarGridSpec(
            num_scalar_prefetch=2,
            grid=(_NP + 1,),
            in_specs=[
                pl.BlockSpec((1, 3, _TB, _H, _W),
                             lambda i, bb, tsp: (jnp.minimum(i, _NP - 1) // npool,
                                                 0,
                                                 jax.lax.rem(
                                                     jnp.minimum(i, _NP - 1),
                                                     npool), 0, 0)),
                pl.BlockSpec(memory_space=pltpu.VMEM),
                pl.BlockSpec((_HD, _C),
                             lambda i, bb, tsp: (jnp.minimum(i, _NCS - 1), 0)),
                pl.BlockSpec(memory_space=pltpu.VMEM),
                pl.BlockSpec(memory_space=pltpu.VMEM),
                pl.BlockSpec(memory_space=pltpu.VMEM),
                pl.BlockSpec(memory_space=pltpu.VMEM),
                pl.BlockSpec(memory_space=pltpu.VMEM),
                pl.BlockSpec(memory_space=pltpu.VMEM),
                pl.BlockSpec(memory_space=pltpu.VMEM),
                pl.BlockSpec((_CQ, _C),
                             lambda i, bb, tsp: (jnp.minimum(i, _NCQ - 1), 0)),
                pl.BlockSpec((_HD, _C),
                             lambda i, bb, tsp: (jnp.minimum(i, _NCS - 1), 0)),
                pl.BlockSpec(memory_space=pltpu.VMEM),
                pl.BlockSpec((_HD, _C),
                             lambda i, bb, tsp: (jnp.minimum(i, _NCS - 1), 0)),
                pl.BlockSpec(memory_space=pltpu.VMEM),
            ],
            out_specs=[
                pl.BlockSpec((1, _TB, 3, 128),
                             lambda i, bb, tsp: (jnp.minimum(i, _NP - 1) // npool,
                                                 jax.lax.rem(
                                                     jnp.minimum(i, _NP - 1),
                                                     npool), 0, 0)),
                pl.BlockSpec(memory_space=pltpu.VMEM),
                pl.BlockSpec(memory_space=pltpu.VMEM),
            ],
            scratch_shapes=[
                pltpu.VMEM((3 * _C, _C), jnp.bfloat16),
                pltpu.VMEM((_C, _C), jnp.bfloat16),
                pltpu.VMEM((_C, _C), jnp.bfloat16),
                pltpu.VMEM((_B * _T, _C), jnp.float32),
                pltpu.VMEM((_B * _NG, _C), jnp.float32),
                pltpu.VMEM((_B * _T, _C), jnp.float32),
            ],
        ),
        compiler_params=pltpu.CompilerParams(
            dimension_semantics=("arbitrary",),
            vmem_limit_bytes=52 * 1024 * 1024,
        ),
        name="fused_vit",
    )(bb_flat, timestamp, x, feats, w_reduce, brd, te, wnr, bnm, cls2,
      lnw, lnb, w_qkv, w_proj, bpj, w_tfc, btf)

    return pool_out[:, :, :, 0], g, pooled


# speedup vs baseline: 1.0402x; 1.0402x over previous
import jax
import jax.numpy as jnp
from jax.experimental import pallas as pl
from jax.experimental.pallas import tpu as pltpu

_EPS = 1e-5
_B = 8
_T = 20
_H = 224
_W = 224
_C = 1536
_NH = 12
_HD = 128
_NG = 5
_K = 5
_TB = 4
_NP = (_B // 2) * (_T // _TB)
_CQ = 288
_NCQ = 3 * _C // _CQ
_NCS = 12
_IT0 = 17


def _attn_iter(cl_ref, gr_ref, wqkv_bf, wproj_bf, wtfc_bf,
               lnw, lnb, bp_ref, bt_ref):
    ncls = _B * _NG
    ntok = _B * _T + ncls
    ri = jax.lax.broadcasted_iota(jnp.int32, (ntok, ntok), 0)
    ci = jax.lax.broadcasted_iota(jnp.int32, (ntok, ntok), 1)
    gid_r = jnp.where(ri < ncls, ri, jax.lax.div(ri - ncls, 4))
    gid_c = jnp.where(ci < ncls, ci, jax.lax.div(ci - ncls, 4))
    mf = (gid_r == gid_c).astype(jnp.float32)
    scale = float(_HD) ** -0.5
    ones_col = jnp.ones((ntok, 1), jnp.float32)
    tok = jnp.concatenate([cl_ref[...], gr_ref[...]], axis=0)
    mu = jnp.mean(tok, axis=1, keepdims=True)
    d = tok - mu
    var = jnp.mean(d * d, axis=1, keepdims=True)
    h = ((d * jax.lax.rsqrt(var + _EPS)) * lnw + lnb)
    qkv = jax.lax.dot_general(
        h.astype(jnp.bfloat16), wqkv_bf[...], (((1,), (1,)), ((), ())),
        preferred_element_type=jnp.float32)
    oparts = []
    for hh in range(_NH):
        qh = qkv[:, hh * _HD:(hh + 1) * _HD]
        kh = qkv[:, _C + hh * _HD:_C + (hh + 1) * _HD]
        vh = qkv[:, 2 * _C + hh * _HD:2 * _C + (hh + 1) * _HD]
        s = jax.lax.dot_general(
            qh, kh, (((1,), (1,)), ((), ())),
            preferred_element_type=jnp.float32)
        e = jnp.exp(s * scale) * mf
        vaug = jnp.concatenate([vh, ones_col], axis=1)
        oa = jax.lax.dot_general(
            e, vaug, (((1,), (0,)), ((), ())),
            preferred_element_type=jnp.float32)
        oparts.append(oa[:, :_HD] * (1.0 / oa[:, _HD:_HD + 1]))
    o = jnp.concatenate(oparts, axis=1)
    p = jax.lax.dot_general(
        o.astype(jnp.bfloat16), wproj_bf[...], (((1,), (1,)), ((), ())),
        preferred_element_type=jnp.float32) + bp_ref[...]
    u = jax.lax.dot_general(
        p.astype(jnp.bfloat16), wtfc_bf[...], (((1,), (1,)), ((), ())),
        preferred_element_type=jnp.float32) + bt_ref[...]
    upd = u + tok
    cl_new = upd[:ncls].reshape(_B, _NG, _C)
    cl_ref[...] = jnp.concatenate(
        [cl_new[:, _NG - 1:_NG, :], cl_new[:, :_NG - 1, :]],
        axis=1).reshape(ncls, _C)
    gr_ref[...] = upd[ncls:]


def _mega_kernel(bb_ref, ts_ref, x_ref, x2_ref, f_ref, wr_ref, brd_ref, te_ref,
                 wn_ref, bn_ref, cls_ref, lnw_ref, lnb_ref, wqkv_ref,
                 wproj_ref, bp_ref, wtfc_ref, bt_ref,
                 o_ref, o2_ref, g_ref, pooled_ref,
                 wqkv_bf, wproj_bf, wtfc_bf, xt_s, cl_ref, gr_ref):
    i = pl.program_id(0)

    @pl.when(i < _NP)
    def _pool():
        q = i // (_T // _TB)
        tb = jax.lax.rem(i, _T // _TB)
        ri = jax.lax.broadcasted_iota(jnp.int32, (_H, _W), 0)
        ci = jax.lax.broadcasted_iota(jnp.int32, (_H, _W), 1)
        inv_hw = 1.0 / float(_H * _W)
        for xr, orf, b in ((x_ref, o_ref, q), (x2_ref, o2_ref, q + _B // 2)):
            for tt in range(_TB):
                base = (b * _T + tb * _TB + tt) * 4
                x1 = bb_ref[base]
                y1 = bb_ref[base + 1]
                x2 = bb_ref[base + 2]
                y2 = bb_ref[base + 3]
                mask = (ri >= y1) & (ri < y2) & (ci >= x1) & (ci < x2)
                area = ((x2 - x1) * (y2 - y1)).astype(jnp.float32)
                inv_area = 1.0 / area
                for c in range(3):
                    img = xr[0, c, tt]
                    tot = jnp.sum(img, keepdims=True)
                    crop = jnp.sum(jnp.where(mask, img, 0.0), keepdims=True)
                    val = tot * inv_hw + crop * inv_area
                    orf[0, tt, c] = jnp.broadcast_to(val, (1, 128))[0]

    @pl.when(i < _NCQ)
    def _stream_qkv():
        r = pl.multiple_of(i * _CQ, _CQ)
        wqkv_bf[pl.ds(r, _CQ), :] = wqkv_ref[...].astype(jnp.bfloat16)

    @pl.when(i < _NCS)
    def _stream_sq():
        r = pl.multiple_of(i * _HD, _HD)
        wproj_bf[pl.ds(r, _HD), :] = wproj_ref[...].astype(jnp.bfloat16)
        wtfc_bf[pl.ds(r, _HD), :] = wtfc_ref[...].astype(jnp.bfloat16)
        xt_s[:, pl.ds(r, _HD)] = jax.lax.dot_general(
            f_ref[...].reshape(_B * _T, _C), wr_ref[...],
            (((1,), (1,)), ((), ())),
            preferred_element_type=jnp.float32)

    @pl.when(i == _NCS)
    def _embed():
        ts = jnp.stack([ts_ref[b] for b in range(_B)]).reshape(_B, 1)
        tsadd = ts * wn_ref[...] + bn_ref[...]
        x3 = (xt_s[...].reshape(_B, _T, _C) + te_ref[...][None, :, :]
              + brd_ref[...][None, :, :] + tsadd[:, None, :])
        pooled_ref[...] = (x3[:, _T - 2, :] + x3[:, _T - 1, :]) * (1.0 / 3.0)
        gr_ref[...] = x3.reshape(_B * _T, _C)
        cl_ref[...] = jnp.broadcast_to(cls_ref[...][None, :, :],
                                       (_B, _NG, _C)).reshape(_B * _NG, _C)

    for it in range(4):
        @pl.when(i == _IT0 + it)
        def _iter():
            _attn_iter(cl_ref, gr_ref, wqkv_bf, wproj_bf, wtfc_bf,
                       lnw_ref[...], lnb_ref[...], bp_ref, bt_ref)

    @pl.when(i == _NP)
    def _finish():
        g_ref[:, :, 0:1, :] = cl_ref[...].reshape(_B, _NG, 1, _C)
        g_ref[:, :, 1:5, :] = gr_ref[...].reshape(_B, _NG, 4, _C)


def kernel(x, timestamp, bboxes, feats, w_reduce, b_reduce, w_num, b_num,
           time_embed, cls_token_swap, ln_t_w, ln_t_b, w_qkv, w_proj, b_proj,
           w_tfc, b_tfc):
    bb_flat = bboxes.reshape(-1)
    brd = b_reduce.reshape(1, _C)
    bnm = b_num.reshape(1, _C)
    wnr = w_num.reshape(1, _C)
    te = time_embed.reshape(_T, _C)
    cls2 = cls_token_swap.reshape(_NG, _C)
    lnw = ln_t_w.reshape(1, _C)
    lnb = ln_t_b.reshape(1, _C)
    bpj = b_proj.reshape(1, _C)
    btf = b_tfc.reshape(1, _C)

    npool = _T // _TB
    pool_lo, pool_hi, g, pooled = pl.pallas_call(
        _mega_kernel,
        out_shape=[jax.ShapeDtypeStruct((_B // 2, _T, 3, 128), jnp.float32),
                   jax.ShapeDtypeStruct((_B // 2, _T, 3, 128), jnp.float32),
                   jax.ShapeDtypeStruct((_B, _NG, _K, _C), jnp.float32),
                   jax.ShapeDtypeStruct((_B, _C), jnp.float32)],
        grid_spec=pltpu.PrefetchScalarGridSpec(
            num_scalar_prefetch=2,
            grid=(_NP + 1,),
            in_specs=[
                pl.BlockSpec((1, 3, _TB, _H, _W),
                             lambda i, bb, tsp: (jnp.minimum(i, _NP - 1) // npool,
                                                 0,
                                                 jax.lax.rem(
                                                     jnp.minimum(i, _NP - 1),
                                                     npool), 0, 0)),
                pl.BlockSpec((1, 3, _TB, _H, _W),
                             lambda i, bb, tsp: (jnp.minimum(i, _NP - 1) // npool
                                                 + _B // 2, 0,
                                                 jax.lax.rem(
                                                     jnp.minimum(i, _NP - 1),
                                                     npool), 0, 0)),
                pl.BlockSpec(memory_space=pltpu.VMEM),
                pl.BlockSpec((_HD, _C),
                             lambda i, bb, tsp: (jnp.minimum(i, _NCS - 1), 0)),
                pl.BlockSpec(memory_space=pltpu.VMEM),
                pl.BlockSpec(memory_space=pltpu.VMEM),
                pl.BlockSpec(memory_space=pltpu.VMEM),
                pl.BlockSpec(memory_space=pltpu.VMEM),
                pl.BlockSpec(memory_space=pltpu.VMEM),
                pl.BlockSpec(memory_space=pltpu.VMEM),
                pl.BlockSpec(memory_space=pltpu.VMEM),
                pl.BlockSpec((_CQ, _C),
                             lambda i, bb, tsp: (jnp.minimum(i, _NCQ - 1), 0)),
                pl.BlockSpec((_HD, _C),
                             lambda i, bb, tsp: (jnp.minimum(i, _NCS - 1), 0)),
                pl.BlockSpec(memory_space=pltpu.VMEM),
                pl.BlockSpec((_HD, _C),
                             lambda i, bb, tsp: (jnp.minimum(i, _NCS - 1), 0)),
                pl.BlockSpec(memory_space=pltpu.VMEM),
            ],
            out_specs=[
                pl.BlockSpec((1, _TB, 3, 128),
                             lambda i, bb, tsp: (jnp.minimum(i, _NP - 1) // npool,
                                                 jax.lax.rem(
                                                     jnp.minimum(i, _NP - 1),
                                                     npool), 0, 0)),
                pl.BlockSpec((1, _TB, 3, 128),
                             lambda i, bb, tsp: (jnp.minimum(i, _NP - 1) // npool,
                                                 jax.lax.rem(
                                                     jnp.minimum(i, _NP - 1),
                                                     npool), 0, 0)),
                pl.BlockSpec(memory_space=pltpu.VMEM),
                pl.BlockSpec(memory_space=pltpu.VMEM),
            ],
            scratch_shapes=[
                pltpu.VMEM((3 * _C, _C), jnp.bfloat16),
                pltpu.VMEM((_C, _C), jnp.bfloat16),
                pltpu.VMEM((_C, _C), jnp.bfloat16),
                pltpu.VMEM((_B * _T, _C), jnp.float32),
                pltpu.VMEM((_B * _NG, _C), jnp.float32),
                pltpu.VMEM((_B * _T, _C), jnp.float32),
            ],
        ),
        compiler_params=pltpu.CompilerParams(
            dimension_semantics=("arbitrary",),
            vmem_limit_bytes=56 * 1024 * 1024,
        ),
        name="fused_vit",
    )(bb_flat, timestamp, x, x, feats, w_reduce, brd, te, wnr, bnm, cls2,
      lnw, lnb, w_qkv, w_proj, bpj, w_tfc, btf)

    ssm_q = jnp.concatenate([pool_lo[:, :, :, 0], pool_hi[:, :, :, 0]], axis=0)
    return ssm_q, g, pooled
